# Initial kernel scaffold; baseline (speedup 1.0000x reference)
import jax
import jax.numpy as jnp
from jax.experimental import pallas as pl


def kernel(x1, adj1, x2, adj2, W_gcn, W_gat, a_src, a_dst, W_proj, b_proj):
    raise NotImplementedError("write your pallas kernel here")


# speedup vs baseline: 1.3801x; 1.3801x over previous
import jax
import jax.numpy as jnp
from jax.experimental import pallas as pl
from jax.experimental.pallas import tpu as pltpu

N = 4096
D = 256
R = 512
C = 512
NI = N // R
NJ = N // C


def _body(x1, x2, wgcn, wgat, asrc, adst, wproj, bproj, adj1, adj2,
          z1, z2, h1p, wh2, s, t, acc1, acc2, deg, den, tmax):
    i = pl.program_id(0)
    j = pl.program_id(1)

    @pl.when((i == 0) & (j == 0))
    def _init():
        h1p[:] = jnp.dot(x1[:], wgcn[:], preferred_element_type=jnp.float32)
        wh = jnp.dot(x2[:], wgat[:], preferred_element_type=jnp.float32)
        wh2[:] = wh
        s[:] = jax.lax.dot_general(wh, asrc[:], (((1,), (1,)), ((), ())),
                                   preferred_element_type=jnp.float32)
        tt = jax.lax.dot_general(adst[:], wh, (((1,), (1,)), ((), ())),
                                 preferred_element_type=jnp.float32)
        t[:] = tt
        tmax[0, 0] = jnp.max(tt)

    @pl.when(j == 0)
    def _reset():
        acc1[:] = jnp.zeros_like(acc1)
        acc2[:] = jnp.zeros_like(acc2)
        deg[:] = jnp.zeros_like(deg)
        den[:] = jnp.zeros_like(den)

    a1 = adj1[:]
    deg[:] = deg[:] + jnp.sum(a1, axis=1, keepdims=True)
    acc1[:] = acc1[:] + jnp.dot(a1, h1p[pl.ds(j * C, C), :],
                                preferred_element_type=jnp.float32)

    s_blk = s[pl.ds(i * R, R), :]
    t_blk = t[:, pl.ds(j * C, C)]
    e = s_blk + t_blk
    e = jnp.where(e > 0, e, 0.2 * e)
    em = s_blk + tmax[0, 0]
    em = jnp.where(em > 0, em, 0.2 * em)
    w = adj2[:] * jnp.exp(e - em)
    den[:] = den[:] + jnp.sum(w, axis=1, keepdims=True)
    acc2[:] = acc2[:] + jnp.dot(w, wh2[pl.ds(j * C, C), :],
                                preferred_element_type=jnp.float32)

    @pl.when(j == NJ - 1)
    def _fin():
        h1 = jnp.maximum(acc1[:] / (deg[:] + 1e-6), 0.0)
        z1[:] = jnp.dot(h1, wproj[:], preferred_element_type=jnp.float32) + bproj[:]
        h2 = acc2[:] / (den[:] + 1e-6)
        h2 = jnp.where(h2 > 0, h2, jnp.exp(jnp.minimum(h2, 0.0)) - 1.0)
        z2[:] = jnp.dot(h2, wproj[:], preferred_element_type=jnp.float32) + bproj[:]


def _call(x1, x2, W_gcn, W_gat, a_src, a_dst, W_proj, b_proj, adj1, adj2,
          interpret=False):
    full = lambda i, j: (0, 0)
    return pl.pallas_call(
        _body,
        grid=(NI, NJ),
        in_specs=[
            pl.BlockSpec((N, D), full),
            pl.BlockSpec((N, D), full),
            pl.BlockSpec((D, D), full),
            pl.BlockSpec((D, D), full),
            pl.BlockSpec((1, D), full),
            pl.BlockSpec((1, D), full),
            pl.BlockSpec((D, D), full),
            pl.BlockSpec((1, D), full),
            pl.BlockSpec((R, C), lambda i, j: (i, j)),
            pl.BlockSpec((R, C), lambda i, j: (i, j)),
        ],
        out_specs=[
            pl.BlockSpec((R, D), lambda i, j: (i, 0)),
            pl.BlockSpec((R, D), lambda i, j: (i, 0)),
        ],
        out_shape=[
            jax.ShapeDtypeStruct((N, D), jnp.float32),
            jax.ShapeDtypeStruct((N, D), jnp.float32),
        ],
        scratch_shapes=[
            pltpu.VMEM((N, D), jnp.float32),
            pltpu.VMEM((N, D), jnp.float32),
            pltpu.VMEM((N, 1), jnp.float32),
            pltpu.VMEM((1, N), jnp.float32),
            pltpu.VMEM((R, D), jnp.float32),
            pltpu.VMEM((R, D), jnp.float32),
            pltpu.VMEM((R, 1), jnp.float32),
            pltpu.VMEM((R, 1), jnp.float32),
            pltpu.SMEM((1, 1), jnp.float32),
        ],
        interpret=interpret,
    )(x1, x2, W_gcn, W_gat, a_src, a_dst, W_proj, b_proj, adj1, adj2)


def kernel(x1, adj1, x2, adj2, W_gcn, W_gat, a_src, a_dst, W_proj, b_proj):
    z1, z2 = _call(x1, x2, W_gcn, W_gat,
                   a_src.reshape(1, D), a_dst.reshape(1, D),
                   W_proj, b_proj.reshape(1, D), adj1, adj2)
    return (z1, z2)


# speedup vs baseline: 1.3816x; 1.0011x over previous
import jax
import jax.numpy as jnp
from jax.experimental import pallas as pl
from jax.experimental.pallas import tpu as pltpu

N = 4096
D = 256
R = 512
C = 512
NI = N // R
NJ = N // C


def _body(x1, x2, wgcn, wgat, asrc, adst, wproj, bproj, adj1, adj2,
          z1, z2, h1p, wh2, s, t, acc1, acc2, deg, den, tmax):
    i = pl.program_id(0)
    j = pl.program_id(1)

    @pl.when((i == 0) & (j == 0))
    def _init():
        h1p[:] = jnp.dot(x1[:], wgcn[:],
                         preferred_element_type=jnp.float32).astype(jnp.bfloat16)
        wh = jnp.dot(x2[:], wgat[:], preferred_element_type=jnp.float32)
        wh2[:] = wh.astype(jnp.bfloat16)
        s[:] = jax.lax.dot_general(wh, asrc[:], (((1,), (1,)), ((), ())),
                                   preferred_element_type=jnp.float32)
        tt = jax.lax.dot_general(adst[:], wh, (((1,), (1,)), ((), ())),
                                 preferred_element_type=jnp.float32)
        t[:] = tt
        tmax[0, 0] = jnp.max(tt)

    @pl.when(j == 0)
    def _reset():
        acc1[:] = jnp.zeros_like(acc1)
        acc2[:] = jnp.zeros_like(acc2)
        deg[:] = jnp.zeros_like(deg)
        den[:] = jnp.zeros_like(den)

    a1 = adj1[:]
    deg[:] = deg[:] + jnp.sum(a1, axis=1, keepdims=True)
    acc1[:] = acc1[:] + jnp.dot(a1.astype(jnp.bfloat16), h1p[pl.ds(j * C, C), :],
                                preferred_element_type=jnp.float32)

    s_blk = s[pl.ds(i * R, R), :]
    t_blk = t[:, pl.ds(j * C, C)]
    e = s_blk + t_blk
    e = jnp.where(e > 0, e, 0.2 * e)
    em = s_blk + tmax[0, 0]
    em = jnp.where(em > 0, em, 0.2 * em)
    w = adj2[:] * jnp.exp(e - em)
    den[:] = den[:] + jnp.sum(w, axis=1, keepdims=True)
    acc2[:] = acc2[:] + jnp.dot(w.astype(jnp.bfloat16), wh2[pl.ds(j * C, C), :],
                                preferred_element_type=jnp.float32)

    @pl.when(j == NJ - 1)
    def _fin():
        h1 = jnp.maximum(acc1[:] / (deg[:] + 1e-6), 0.0)
        z1[:] = jnp.dot(h1, wproj[:], preferred_element_type=jnp.float32) + bproj[:]
        h2 = acc2[:] / (den[:] + 1e-6)
        h2 = jnp.where(h2 > 0, h2, jnp.exp(jnp.minimum(h2, 0.0)) - 1.0)
        z2[:] = jnp.dot(h2, wproj[:], preferred_element_type=jnp.float32) + bproj[:]


def _call(x1, x2, W_gcn, W_gat, a_src, a_dst, W_proj, b_proj, adj1, adj2,
          interpret=False):
    full = lambda i, j: (0, 0)
    return pl.pallas_call(
        _body,
        grid=(NI, NJ),
        in_specs=[
            pl.BlockSpec((N, D), full),
            pl.BlockSpec((N, D), full),
            pl.BlockSpec((D, D), full),
            pl.BlockSpec((D, D), full),
            pl.BlockSpec((1, D), full),
            pl.BlockSpec((1, D), full),
            pl.BlockSpec((D, D), full),
            pl.BlockSpec((1, D), full),
            pl.BlockSpec((R, C), lambda i, j: (i, j)),
            pl.BlockSpec((R, C), lambda i, j: (i, j)),
        ],
        out_specs=[
            pl.BlockSpec((R, D), lambda i, j: (i, 0)),
            pl.BlockSpec((R, D), lambda i, j: (i, 0)),
        ],
        out_shape=[
            jax.ShapeDtypeStruct((N, D), jnp.float32),
            jax.ShapeDtypeStruct((N, D), jnp.float32),
        ],
        scratch_shapes=[
            pltpu.VMEM((N, D), jnp.bfloat16),
            pltpu.VMEM((N, D), jnp.bfloat16),
            pltpu.VMEM((N, 1), jnp.float32),
            pltpu.VMEM((1, N), jnp.float32),
            pltpu.VMEM((R, D), jnp.float32),
            pltpu.VMEM((R, D), jnp.float32),
            pltpu.VMEM((R, 1), jnp.float32),
            pltpu.VMEM((R, 1), jnp.float32),
            pltpu.SMEM((1, 1), jnp.float32),
        ],
        interpret=interpret,
    )(x1, x2, W_gcn, W_gat, a_src, a_dst, W_proj, b_proj, adj1, adj2)


def kernel(x1, adj1, x2, adj2, W_gcn, W_gat, a_src, a_dst, W_proj, b_proj):
    z1, z2 = _call(x1, x2, W_gcn, W_gat,
                   a_src.reshape(1, D), a_dst.reshape(1, D),
                   W_proj, b_proj.reshape(1, D), adj1, adj2)
    return (z1, z2)


# speedup vs baseline: 1.4038x; 1.0161x over previous
import jax
import jax.numpy as jnp
from jax.experimental import pallas as pl
from jax.experimental.pallas import tpu as pltpu

N = 4096
D = 256
R = 512
C = 512
NI = N // R
NJ = N // C


def _body(x1, x2, wgcn, wgat, asrc, adst, wproj, bproj, adj1, adj2,
          z1, z2, h1p, wh2, s, es1, es2, et1, et2, acc1, acc2, deg, den, tmax):
    i = pl.program_id(0)
    j = pl.program_id(1)

    @pl.when((i == 0) & (j == 0))
    def _init():
        h1p[:] = jnp.dot(x1[:], wgcn[:],
                         preferred_element_type=jnp.float32).astype(jnp.bfloat16)
        wh = jnp.dot(x2[:], wgat[:], preferred_element_type=jnp.float32)
        wh2[:] = wh.astype(jnp.bfloat16)
        ss = jax.lax.dot_general(wh, asrc[:], (((1,), (1,)), ((), ())),
                                 preferred_element_type=jnp.float32)
        s[:] = ss
        es1[:] = jnp.exp(ss)
        es2[:] = jnp.exp(0.2 * ss)
        tt = jax.lax.dot_general(adst[:], wh, (((1,), (1,)), ((), ())),
                                 preferred_element_type=jnp.float32)
        et1[:] = jnp.exp(tt)
        et2[:] = jnp.exp(0.2 * tt)
        tmax[0, 0] = jnp.max(tt)

    @pl.when(j == 0)
    def _reset():
        acc1[:] = jnp.zeros_like(acc1)
        acc2[:] = jnp.zeros_like(acc2)
        deg[:] = jnp.zeros_like(deg)
        den[:] = jnp.zeros_like(den)

    a1 = adj1[:]
    deg[:] = deg[:] + jnp.sum(a1, axis=1, keepdims=True)
    acc1[:] = acc1[:] + jnp.dot(a1.astype(jnp.bfloat16), h1p[pl.ds(j * C, C), :],
                                preferred_element_type=jnp.float32)

    p1 = es1[pl.ds(i * R, R), :] * et1[:, pl.ds(j * C, C)]
    p2 = es2[pl.ds(i * R, R), :] * et2[:, pl.ds(j * C, C)]
    w = adj2[:] * jnp.maximum(p1, p2)
    den[:] = den[:] + jnp.sum(w, axis=1, keepdims=True)
    acc2[:] = acc2[:] + jnp.dot(w.astype(jnp.bfloat16), wh2[pl.ds(j * C, C), :],
                                preferred_element_type=jnp.float32)

    @pl.when(j == NJ - 1)
    def _fin():
        h1 = jnp.maximum(acc1[:] / (deg[:] + 1e-6), 0.0)
        z1[:] = jnp.dot(h1, wproj[:], preferred_element_type=jnp.float32) + bproj[:]
        em = s[pl.ds(i * R, R), :] + tmax[0, 0]
        em = jnp.where(em > 0, em, 0.2 * em)
        h2 = acc2[:] / (den[:] + 1e-6 * jnp.exp(em))
        h2 = jnp.where(h2 > 0, h2, jnp.exp(jnp.minimum(h2, 0.0)) - 1.0)
        z2[:] = jnp.dot(h2, wproj[:], preferred_element_type=jnp.float32) + bproj[:]


def _call(x1, x2, W_gcn, W_gat, a_src, a_dst, W_proj, b_proj, adj1, adj2,
          interpret=False):
    full = lambda i, j: (0, 0)
    return pl.pallas_call(
        _body,
        grid=(NI, NJ),
        in_specs=[
            pl.BlockSpec((N, D), full),
            pl.BlockSpec((N, D), full),
            pl.BlockSpec((D, D), full),
            pl.BlockSpec((D, D), full),
            pl.BlockSpec((1, D), full),
            pl.BlockSpec((1, D), full),
            pl.BlockSpec((D, D), full),
            pl.BlockSpec((1, D), full),
            pl.BlockSpec((R, C), lambda i, j: (i, j)),
            pl.BlockSpec((R, C), lambda i, j: (i, j)),
        ],
        out_specs=[
            pl.BlockSpec((R, D), lambda i, j: (i, 0)),
            pl.BlockSpec((R, D), lambda i, j: (i, 0)),
        ],
        out_shape=[
            jax.ShapeDtypeStruct((N, D), jnp.float32),
            jax.ShapeDtypeStruct((N, D), jnp.float32),
        ],
        scratch_shapes=[
            pltpu.VMEM((N, D), jnp.bfloat16),
            pltpu.VMEM((N, D), jnp.bfloat16),
            pltpu.VMEM((N, 1), jnp.float32),
            pltpu.VMEM((N, 1), jnp.float32),
            pltpu.VMEM((N, 1), jnp.float32),
            pltpu.VMEM((1, N), jnp.float32),
            pltpu.VMEM((1, N), jnp.float32),
            pltpu.VMEM((R, D), jnp.float32),
            pltpu.VMEM((R, D), jnp.float32),
            pltpu.VMEM((R, 1), jnp.float32),
            pltpu.VMEM((R, 1), jnp.float32),
            pltpu.SMEM((1, 1), jnp.float32),
        ],
        interpret=interpret,
    )(x1, x2, W_gcn, W_gat, a_src, a_dst, W_proj, b_proj, adj1, adj2)


def kernel(x1, adj1, x2, adj2, W_gcn, W_gat, a_src, a_dst, W_proj, b_proj):
    z1, z2 = _call(x1, x2, W_gcn, W_gat,
                   a_src.reshape(1, D), a_dst.reshape(1, D),
                   W_proj, b_proj.reshape(1, D), adj1, adj2)
    return (z1, z2)


# speedup vs baseline: 2.1359x; 1.5215x over previous
import jax
import jax.numpy as jnp
from jax.experimental import pallas as pl
from jax.experimental.pallas import tpu as pltpu

N = 4096
D = 256
R = 1024
C = 1024
NI = N // R
NJ = N // C


def _body(x1, x2, wgcn, wgat, asrc, adst, wproj, bproj, adj1, adj2,
          z1, z2, h1p, wh2, s, es1, es2, et1, et2, acc1, acc2, deg, den, tmax):
    i = pl.program_id(0)
    j = pl.program_id(1)

    @pl.when((i == 0) & (j == 0))
    def _init():
        h1p[:] = jnp.dot(x1[:], wgcn[:],
                         preferred_element_type=jnp.float32).astype(jnp.bfloat16)
        wh = jnp.dot(x2[:], wgat[:], preferred_element_type=jnp.float32)
        wh2[:] = wh.astype(jnp.bfloat16)
        ss = jax.lax.dot_general(wh, asrc[:], (((1,), (1,)), ((), ())),
                                 preferred_element_type=jnp.float32)
        s[:] = ss
        es1[:] = jnp.exp(ss)
        es2[:] = jnp.exp(0.2 * ss)
        tt = jax.lax.dot_general(adst[:], wh, (((1,), (1,)), ((), ())),
                                 preferred_element_type=jnp.float32)
        et1[:] = jnp.exp(tt)
        et2[:] = jnp.exp(0.2 * tt)
        tmax[0, 0] = jnp.max(tt)

    @pl.when(j == 0)
    def _reset():
        acc1[:] = jnp.zeros_like(acc1)
        acc2[:] = jnp.zeros_like(acc2)
        deg[:] = jnp.zeros_like(deg)
        den[:] = jnp.zeros_like(den)

    a1 = adj1[:]
    deg[:] = deg[:] + jnp.sum(a1, axis=1, keepdims=True)
    acc1[:] = acc1[:] + jnp.dot(a1.astype(jnp.bfloat16), h1p[pl.ds(j * C, C), :],
                                preferred_element_type=jnp.float32)

    p1 = es1[pl.ds(i * R, R), :] * et1[:, pl.ds(j * C, C)]
    p2 = es2[pl.ds(i * R, R), :] * et2[:, pl.ds(j * C, C)]
    w = adj2[:] * jnp.maximum(p1, p2)
    den[:] = den[:] + jnp.sum(w, axis=1, keepdims=True)
    acc2[:] = acc2[:] + jnp.dot(w.astype(jnp.bfloat16), wh2[pl.ds(j * C, C), :],
                                preferred_element_type=jnp.float32)

    @pl.when(j == NJ - 1)
    def _fin():
        h1 = jnp.maximum(acc1[:] / (deg[:] + 1e-6), 0.0)
        z1[:] = jnp.dot(h1, wproj[:], preferred_element_type=jnp.float32) + bproj[:]
        em = s[pl.ds(i * R, R), :] + tmax[0, 0]
        em = jnp.where(em > 0, em, 0.2 * em)
        h2 = acc2[:] / (den[:] + 1e-6 * jnp.exp(em))
        h2 = jnp.where(h2 > 0, h2, jnp.exp(jnp.minimum(h2, 0.0)) - 1.0)
        z2[:] = jnp.dot(h2, wproj[:], preferred_element_type=jnp.float32) + bproj[:]


def _call(x1, x2, W_gcn, W_gat, a_src, a_dst, W_proj, b_proj, adj1, adj2,
          interpret=False):
    full = lambda i, j: (0, 0)
    return pl.pallas_call(
        _body,
        grid=(NI, NJ),
        in_specs=[
            pl.BlockSpec((N, D), full),
            pl.BlockSpec((N, D), full),
            pl.BlockSpec((D, D), full),
            pl.BlockSpec((D, D), full),
            pl.BlockSpec((1, D), full),
            pl.BlockSpec((1, D), full),
            pl.BlockSpec((D, D), full),
            pl.BlockSpec((1, D), full),
            pl.BlockSpec((R, C), lambda i, j: (i, j)),
            pl.BlockSpec((R, C), lambda i, j: (i, j)),
        ],
        out_specs=[
            pl.BlockSpec((R, D), lambda i, j: (i, 0)),
            pl.BlockSpec((R, D), lambda i, j: (i, 0)),
        ],
        out_shape=[
            jax.ShapeDtypeStruct((N, D), jnp.float32),
            jax.ShapeDtypeStruct((N, D), jnp.float32),
        ],
        scratch_shapes=[
            pltpu.VMEM((N, D), jnp.bfloat16),
            pltpu.VMEM((N, D), jnp.bfloat16),
            pltpu.VMEM((N, 1), jnp.float32),
            pltpu.VMEM((N, 1), jnp.float32),
            pltpu.VMEM((N, 1), jnp.float32),
            pltpu.VMEM((1, N), jnp.float32),
            pltpu.VMEM((1, N), jnp.float32),
            pltpu.VMEM((R, D), jnp.float32),
            pltpu.VMEM((R, D), jnp.float32),
            pltpu.VMEM((R, 1), jnp.float32),
            pltpu.VMEM((R, 1), jnp.float32),
            pltpu.SMEM((1, 1), jnp.float32),
        ],
        interpret=interpret,
    )(x1, x2, W_gcn, W_gat, a_src, a_dst, W_proj, b_proj, adj1, adj2)


def kernel(x1, adj1, x2, adj2, W_gcn, W_gat, a_src, a_dst, W_proj, b_proj):
    z1, z2 = _call(x1, x2, W_gcn, W_gat,
                   a_src.reshape(1, D), a_dst.reshape(1, D),
                   W_proj, b_proj.reshape(1, D), adj1, adj2)
    return (z1, z2)
